# baseline (speedup 1.0000x reference)
import jax
import jax.numpy as jnp
from jax import lax
from jax.experimental import pallas as pl

N, D = 100000, 64
N2 = N // 2
BR = 5000
G = N2 // BR


def _body(ce_ref, co_ref, w_ref, b_ref, tab_ref, out_ref):
    w = w_ref[...]
    oe = lax.dot_general(ce_ref[0], w, (((0,), (0,)), ((), ())),
                         preferred_element_type=jnp.float32)
    oo = lax.dot_general(co_ref[0], w, (((0,), (0,)), ((), ())),
                         preferred_element_type=jnp.float32)
    t = tab_ref[...] + b_ref[...]
    out_ref[:, 0:D] = t[:, 0:D] + oe
    out_ref[:, D:2 * D] = t[:, D:2 * D] + oo


def kernel(initial_concentration, W_conc, b_conc, identity_table):
    table128 = identity_table.reshape(N2, 2 * D)
    ce = initial_concentration[0::2].reshape(G, 1, BR)
    co = initial_concentration[1::2].reshape(G, 1, BR)
    w = W_conc.reshape(1, D)
    b = jnp.tile(b_conc, 2).reshape(1, 2 * D)
    out = pl.pallas_call(
        _body,
        grid=(G,),
        in_specs=[
            pl.BlockSpec((1, 1, BR), lambda i: (i, 0, 0)),
            pl.BlockSpec((1, 1, BR), lambda i: (i, 0, 0)),
            pl.BlockSpec((1, D), lambda i: (0, 0)),
            pl.BlockSpec((1, 2 * D), lambda i: (0, 0)),
            pl.BlockSpec((BR, 2 * D), lambda i: (i, 0)),
        ],
        out_specs=pl.BlockSpec((BR, 2 * D), lambda i: (i, 0)),
        out_shape=jax.ShapeDtypeStruct((N2, 2 * D), jnp.float32),
    )(ce, co, w, b, table128)
    return out.reshape(N, D)


# speedup vs baseline: 1.3194x; 1.3194x over previous
import jax
import jax.numpy as jnp
from jax import lax
from jax.experimental import pallas as pl
from jax.experimental.pallas import tpu as pltpu

N, D = 100000, 64
BR = 4096
NCH = (N + BR - 1) // BR
LAST = N - (NCH - 1) * BR


def _rows(i):
    return LAST if i == NCH - 1 else BR


def _body(conc_hbm, w_hbm, b_hbm, tab_hbm, out_hbm,
          buf0, buf1, cb0, cb1, wb, bb,
          ls0, ls1, cs0, cs1, ss0, ss1):
    pltpu.make_async_copy(w_hbm, wb, ls0).start()
    pltpu.make_async_copy(b_hbm, bb, ls1).start()
    pltpu.make_async_copy(w_hbm, wb, ls0).wait()
    pltpu.make_async_copy(b_hbm, bb, ls1).wait()

    bufs, cbs = [buf0, buf1], [cb0, cb1]
    lsems, csems, ssems = [ls0, ls1], [cs0, cs1], [ss0, ss1]

    def load(i):
        p = i % 2
        base = i * BR
        r = _rows(i)
        pltpu.make_async_copy(tab_hbm.at[pl.ds(base, r)], bufs[p].at[pl.ds(0, r)],
                              lsems[p]).start()
        pltpu.make_async_copy(conc_hbm.at[pl.ds(base, BR)], cbs[p], csems[p]).start()

    def compute_store(i):
        p = i % 2
        base = i * BR
        r = _rows(i)
        pltpu.make_async_copy(tab_hbm.at[pl.ds(base, r)], bufs[p].at[pl.ds(0, r)],
                              lsems[p]).wait()
        pltpu.make_async_copy(conc_hbm.at[pl.ds(base, BR)], cbs[p], csems[p]).wait()
        cm = cbs[p][...].reshape(1, BR)
        outer = lax.dot_general(cm, wb[...], (((0,), (0,)), ((), ())),
                                preferred_element_type=jnp.float32)
        bufs[p][...] = bufs[p][...] + outer + bb[...]
        pltpu.make_async_copy(bufs[p].at[pl.ds(0, r)], out_hbm.at[pl.ds(base, r)],
                              ssems[p]).start()

    def wait_store(i):
        p = i % 2
        base = i * BR
        r = _rows(i)
        pltpu.make_async_copy(bufs[p].at[pl.ds(0, r)], out_hbm.at[pl.ds(base, r)],
                              ssems[p]).wait()

    load(0)
    for i in range(NCH):
        if i + 1 < NCH:
            if i >= 1:
                wait_store(i - 1)
            load(i + 1)
        compute_store(i)
    if NCH >= 2:
        wait_store(NCH - 2)
    wait_store(NCH - 1)


def kernel(initial_concentration, W_conc, b_conc, identity_table):
    conc_p = jnp.pad(initial_concentration, (0, NCH * BR - N))
    w = W_conc.reshape(1, D)
    b = b_conc.reshape(1, D)
    out = pl.pallas_call(
        _body,
        in_specs=[
            pl.BlockSpec(memory_space=pl.ANY),
            pl.BlockSpec(memory_space=pl.ANY),
            pl.BlockSpec(memory_space=pl.ANY),
            pl.BlockSpec(memory_space=pl.ANY),
        ],
        out_specs=pl.BlockSpec(memory_space=pl.ANY),
        out_shape=jax.ShapeDtypeStruct((N, D), jnp.float32),
        scratch_shapes=[
            pltpu.VMEM((BR, D), jnp.float32),
            pltpu.VMEM((BR, D), jnp.float32),
            pltpu.VMEM((BR,), jnp.float32),
            pltpu.VMEM((BR,), jnp.float32),
            pltpu.VMEM((1, D), jnp.float32),
            pltpu.VMEM((1, D), jnp.float32),
            pltpu.SemaphoreType.DMA,
            pltpu.SemaphoreType.DMA,
            pltpu.SemaphoreType.DMA,
            pltpu.SemaphoreType.DMA,
            pltpu.SemaphoreType.DMA,
            pltpu.SemaphoreType.DMA,
        ],
    )(conc_p, w, b, identity_table)
    return out


# speedup vs baseline: 1.5867x; 1.2026x over previous
import jax
import jax.numpy as jnp
from jax import lax
from jax.experimental import pallas as pl

N, D = 100000, 64
BR = 8192
G = (N + BR - 1) // BR


def _body(conc_ref, w_ref, b_ref, tab_ref, out_ref):
    cm = conc_ref[...].reshape(1, BR)
    outer = lax.dot_general(cm, w_ref[...], (((0,), (0,)), ((), ())),
                            preferred_element_type=jnp.float32)
    out_ref[...] = tab_ref[...] + outer + b_ref[...]


def kernel(initial_concentration, W_conc, b_conc, identity_table):
    w = W_conc.reshape(1, D)
    b = b_conc.reshape(1, D)
    out = pl.pallas_call(
        _body,
        grid=(G,),
        in_specs=[
            pl.BlockSpec((BR,), lambda i: (i,)),
            pl.BlockSpec((1, D), lambda i: (0, 0)),
            pl.BlockSpec((1, D), lambda i: (0, 0)),
            pl.BlockSpec((BR, D), lambda i: (i, 0)),
        ],
        out_specs=pl.BlockSpec((BR, D), lambda i: (i, 0)),
        out_shape=jax.ShapeDtypeStruct((N, D), jnp.float32),
    )(initial_concentration, w, b, identity_table)
    return out


# speedup vs baseline: 1.6073x; 1.0130x over previous
import jax
import jax.numpy as jnp
from jax import lax
from jax.experimental import pallas as pl

N, D = 100000, 64
BR = 16384
G = (N + BR - 1) // BR


def _body(conc_ref, w_ref, b_ref, tab_ref, out_ref):
    cm = conc_ref[...].reshape(1, BR)
    outer = lax.dot_general(cm, w_ref[...], (((0,), (0,)), ((), ())),
                            preferred_element_type=jnp.float32)
    out_ref[...] = tab_ref[...] + outer + b_ref[...]


def kernel(initial_concentration, W_conc, b_conc, identity_table):
    w = W_conc.reshape(1, D)
    b = b_conc.reshape(1, D)
    out = pl.pallas_call(
        _body,
        grid=(G,),
        in_specs=[
            pl.BlockSpec((BR,), lambda i: (i,)),
            pl.BlockSpec((1, D), lambda i: (0, 0)),
            pl.BlockSpec((1, D), lambda i: (0, 0)),
            pl.BlockSpec((BR, D), lambda i: (i, 0)),
        ],
        out_specs=pl.BlockSpec((BR, D), lambda i: (i, 0)),
        out_shape=jax.ShapeDtypeStruct((N, D), jnp.float32),
    )(initial_concentration, w, b, identity_table)
    return out
